# baseline (speedup 1.0000x reference)
import functools

import jax
import jax.numpy as jnp
from jax import lax
from jax.experimental import pallas as pl
from jax.experimental.pallas import tpu as pltpu
from jax.experimental.pallas import tpu_sc as plsc

N = 6400000
K = 1000
KP = 1024
NC = 2
NS = 16
NW = NC * NS
H = 2
NPTS = N // H
P = NPTS // NW
T = 2000
NT = P // T
SUB = T // 16
UF = 5
LANES = 16
TBL = LANES * 4 * KP


def _sc_segment_sums(dx, dy, dz, labels, base):
    mesh = plsc.VectorSubcoreMesh(core_axis_name="c", subcore_axis_name="s")

    @functools.partial(
        pl.kernel,
        mesh=mesh,
        out_type=jax.ShapeDtypeStruct((NW * 4, KP), jnp.float32),
        scratch_types=[
            pltpu.VMEM((TBL,), jnp.float32),
            pltpu.VMEM((4, KP), jnp.float32),
            pltpu.VMEM((T,), jnp.int32),
            pltpu.VMEM((T,), jnp.float32),
            pltpu.VMEM((T,), jnp.float32),
            pltpu.VMEM((T,), jnp.float32),
            pltpu.VMEM((T,), jnp.int32),
            pltpu.VMEM((T,), jnp.float32),
            pltpu.VMEM((T,), jnp.float32),
            pltpu.VMEM((T,), jnp.float32),
            pltpu.SemaphoreType.DMA,
            pltpu.SemaphoreType.DMA,
        ],
        compiler_params=pltpu.CompilerParams(needs_layout_passes=False),
    )
    def k(dx_hbm, dy_hbm, dz_hbm, lab_hbm, out_hbm,
          table, res, lab_v0, dx_v0, dy_v0, dz_v0,
          lab_v1, dx_v1, dy_v1, dz_v1, sem0, sem1):
        wid = lax.axis_index("s") * NC + lax.axis_index("c")
        lane = lax.iota(jnp.int32, 16)
        ivec = lane * SUB
        lanebase = lane * (4 * KP)
        zeros = jnp.zeros((16,), jnp.float32)
        ones = jnp.ones((16,), jnp.float32)
        izeros = jnp.zeros((16,), jnp.int32)
        bufs = ((lab_v0, dx_v0, dy_v0, dz_v0), (lab_v1, dx_v1, dy_v1, dz_v1))
        sems = (sem0, sem1)

        pbase = wid * P

        def start(t, b):
            off = pl.ds(pbase + t * T, T)
            loff = pl.ds(base + pbase + t * T, T)
            lab_b, dx_b, dy_b, dz_b = bufs[b]
            sem = sems[b]
            pltpu.async_copy(lab_hbm.at[loff], lab_b, sem)
            pltpu.async_copy(dx_hbm.at[off], dx_b, sem)
            pltpu.async_copy(dy_hbm.at[off], dy_b, sem)
            pltpu.async_copy(dz_hbm.at[off], dz_b, sem)

        cur = (izeros, zeros, zeros, zeros, zeros)
        start(0, 0)

        def zero_body(i, _):
            for u in range(8):
                table[pl.ds((i * 8 + u) * 16, 16)] = zeros
            return 0

        lax.fori_loop(0, TBL // 128, zero_body, 0)
        start(1, 1)

        def make_step(lab_b, dx_b, dy_b, dz_b):
            def step(s, carry):
                clab, ax, ay, az, ac = carry
                pidx = ivec + s
                nlab = plsc.load_gather(lab_b, [pidx])
                vx = plsc.load_gather(dx_b, [pidx])
                vy = plsc.load_gather(dy_b, [pidx])
                vz = plsc.load_gather(dz_b, [pidx])
                flush = nlab != clab
                bi = lanebase + clab
                plsc.addupdate_scatter(table, [bi], ax, mask=flush)
                plsc.addupdate_scatter(table, [bi + KP], ay, mask=flush)
                plsc.addupdate_scatter(table, [bi + 2 * KP], az, mask=flush)
                plsc.addupdate_scatter(table, [bi + 3 * KP], ac, mask=flush)
                ax = jnp.where(flush, vx, ax + vx)
                ay = jnp.where(flush, vy, ay + vy)
                az = jnp.where(flush, vz, az + vz)
                ac = jnp.where(flush, 1.0, ac + 1.0)
                return (nlab, ax, ay, az, ac)

            def stepu(i, carry):
                for u in range(UF):
                    carry = step(i * UF + u, carry)
                return carry

            return stepu

        def pair(i, carry):
            for b in range(2):
                t = 2 * i + b
                lab_b, dx_b, dy_b, dz_b = bufs[b]
                sem = sems[b]
                pltpu.make_async_copy(lab_hbm.at[pl.ds(0, T)], lab_b, sem).wait()
                pltpu.make_async_copy(dx_hbm.at[pl.ds(0, T)], dx_b, sem).wait()
                pltpu.make_async_copy(dy_hbm.at[pl.ds(0, T)], dy_b, sem).wait()
                pltpu.make_async_copy(dz_hbm.at[pl.ds(0, T)], dz_b, sem).wait()
                carry = lax.fori_loop(
                    0, SUB // UF, make_step(lab_b, dx_b, dy_b, dz_b), carry
                )

                @pl.when(t + 2 < NT)
                def _():
                    start(t + 2, b)

            return carry

        cur = lax.fori_loop(0, NT // 2, pair, cur)

        clab, ax, ay, az, ac = cur
        bi = lanebase + clab
        plsc.addupdate_scatter(table, [bi], ax)
        plsc.addupdate_scatter(table, [bi + KP], ay)
        plsc.addupdate_scatter(table, [bi + 2 * KP], az)
        plsc.addupdate_scatter(table, [bi + 3 * KP], ac)

        for c in range(4):
            def red_body(j, _, c=c):
                acc = zeros
                for l in range(LANES):
                    acc = acc + table[pl.ds(l * (4 * KP) + c * KP + j * 16, 16)]
                res[c, pl.ds(j * 16, 16)] = acc
                return 0

            lax.fori_loop(0, KP // 16, red_body, 0)

        pltpu.sync_copy(res, out_hbm.at[pl.ds(wid * 4, 4)])

    return k(dx, dy, dz, labels)


def _loss_body(pa_ref, pb_ref, o_ref):
    x = pa_ref[...] + pb_ref[...]
    rid = lax.broadcasted_iota(jnp.int32, (NW * 4, KP), 0)
    rmod = lax.rem(rid, 4)

    def csum(c):
        return jnp.sum(jnp.where(rmod == c, x, 0.0), axis=0, keepdims=True)

    s0, s1, s2, cnt = csum(0), csum(1), csum(2), csum(3)
    safe = jnp.where(cnt > 0.0, cnt, 1.0)
    d0, d1, d2 = s0 / safe, s1 / safe, s2 / safe
    pt = d0 * d0 + d1 * d1 + d2 * d2
    kidx = lax.broadcasted_iota(jnp.int32, (1, KP), 1)
    valid = kidx > 0
    contrib = (cnt >= 2.0) & valid
    present = (cnt >= 1.0) & valid
    total = jnp.sum(jnp.where(contrib, pt, 0.0))
    ntree = jnp.sum(jnp.where(present, 1.0, 0.0))
    loss = jnp.where(ntree > 0.0, total / jnp.maximum(ntree, 1.0), 0.0)
    o_ref[...] = jnp.full((1, 1), loss, jnp.float32)


def kernel(coords, offset_inst, offset_tree, tree_labels):
    del coords
    partials = []
    for h in range(H):
        lo = h * NPTS
        oi = jax.lax.slice_in_dim(offset_inst, lo, lo + NPTS, axis=0)
        ot = jax.lax.slice_in_dim(offset_tree, lo, lo + NPTS, axis=0)
        dx = oi[:, 0] - ot[:, 0]
        dy = oi[:, 1] - ot[:, 1]
        dz = oi[:, 2] - ot[:, 2]
        partials.append(_sc_segment_sums(dx, dy, dz, tree_labels, lo))
    loss = pl.pallas_call(
        _loss_body,
        out_shape=jax.ShapeDtypeStruct((1, 1), jnp.float32),
    )(*partials)
    return jnp.reshape(loss, ())


# speedup vs baseline: 1.0126x; 1.0126x over previous
import functools

import jax
import jax.numpy as jnp
from jax import lax
from jax.experimental import pallas as pl
from jax.experimental.pallas import tpu as pltpu
from jax.experimental.pallas import tpu_sc as plsc

N = 6400000
K = 1000
KP = 1024
NC = 2
NS = 16
NW = NC * NS
H = 1
NPTS = N // H
P = NPTS // NW
T = 4000
NT = P // T
SUB = T // 16
UF = 5
LANES = 16
TBL = LANES * 4 * KP


def _sc_segment_sums(dx, dy, dz, labels, base):
    mesh = plsc.VectorSubcoreMesh(core_axis_name="c", subcore_axis_name="s")

    @functools.partial(
        pl.kernel,
        mesh=mesh,
        out_type=jax.ShapeDtypeStruct((NW * 4, KP), jnp.float32),
        scratch_types=[
            pltpu.VMEM((TBL,), jnp.float32),
            pltpu.VMEM((4, KP), jnp.float32),
            pltpu.VMEM((T,), jnp.int32),
            pltpu.VMEM((T,), jnp.float32),
            pltpu.VMEM((T,), jnp.float32),
            pltpu.VMEM((T,), jnp.float32),
            pltpu.VMEM((T,), jnp.int32),
            pltpu.VMEM((T,), jnp.float32),
            pltpu.VMEM((T,), jnp.float32),
            pltpu.VMEM((T,), jnp.float32),
            pltpu.SemaphoreType.DMA,
            pltpu.SemaphoreType.DMA,
        ],
        compiler_params=pltpu.CompilerParams(needs_layout_passes=False),
    )
    def k(dx_hbm, dy_hbm, dz_hbm, lab_hbm, out_hbm,
          table, res, lab_v0, dx_v0, dy_v0, dz_v0,
          lab_v1, dx_v1, dy_v1, dz_v1, sem0, sem1):
        wid = lax.axis_index("s") * NC + lax.axis_index("c")
        lane = lax.iota(jnp.int32, 16)
        ivec = lane * SUB
        lanebase = lane * (4 * KP)
        zeros = jnp.zeros((16,), jnp.float32)
        ones = jnp.ones((16,), jnp.float32)
        izeros = jnp.zeros((16,), jnp.int32)
        bufs = ((lab_v0, dx_v0, dy_v0, dz_v0), (lab_v1, dx_v1, dy_v1, dz_v1))
        sems = (sem0, sem1)

        pbase = wid * P

        def start(t, b):
            off = pl.ds(pbase + t * T, T)
            loff = pl.ds(base + pbase + t * T, T)
            lab_b, dx_b, dy_b, dz_b = bufs[b]
            sem = sems[b]
            pltpu.async_copy(lab_hbm.at[loff], lab_b, sem)
            pltpu.async_copy(dx_hbm.at[off], dx_b, sem)
            pltpu.async_copy(dy_hbm.at[off], dy_b, sem)
            pltpu.async_copy(dz_hbm.at[off], dz_b, sem)

        cur = (izeros, zeros, zeros, zeros, zeros)
        start(0, 0)

        def zero_body(i, _):
            for u in range(8):
                table[pl.ds((i * 8 + u) * 16, 16)] = zeros
            return 0

        lax.fori_loop(0, TBL // 128, zero_body, 0)
        start(1, 1)

        def make_step(lab_b, dx_b, dy_b, dz_b):
            def step(s, carry):
                clab, ax, ay, az, ac = carry
                pidx = ivec + s
                nlab = plsc.load_gather(lab_b, [pidx])
                vx = plsc.load_gather(dx_b, [pidx])
                vy = plsc.load_gather(dy_b, [pidx])
                vz = plsc.load_gather(dz_b, [pidx])
                flush = nlab != clab
                bi = lanebase + clab
                plsc.addupdate_scatter(table, [bi], ax, mask=flush)
                plsc.addupdate_scatter(table, [bi + KP], ay, mask=flush)
                plsc.addupdate_scatter(table, [bi + 2 * KP], az, mask=flush)
                plsc.addupdate_scatter(table, [bi + 3 * KP], ac, mask=flush)
                ax = jnp.where(flush, vx, ax + vx)
                ay = jnp.where(flush, vy, ay + vy)
                az = jnp.where(flush, vz, az + vz)
                ac = jnp.where(flush, 1.0, ac + 1.0)
                return (nlab, ax, ay, az, ac)

            def stepu(i, carry):
                for u in range(UF):
                    carry = step(i * UF + u, carry)
                return carry

            return stepu

        def pair(i, carry):
            for b in range(2):
                t = 2 * i + b
                lab_b, dx_b, dy_b, dz_b = bufs[b]
                sem = sems[b]
                pltpu.make_async_copy(lab_hbm.at[pl.ds(0, T)], lab_b, sem).wait()
                pltpu.make_async_copy(dx_hbm.at[pl.ds(0, T)], dx_b, sem).wait()
                pltpu.make_async_copy(dy_hbm.at[pl.ds(0, T)], dy_b, sem).wait()
                pltpu.make_async_copy(dz_hbm.at[pl.ds(0, T)], dz_b, sem).wait()
                carry = lax.fori_loop(
                    0, SUB // UF, make_step(lab_b, dx_b, dy_b, dz_b), carry
                )

                @pl.when(t + 2 < NT)
                def _():
                    start(t + 2, b)

            return carry

        cur = lax.fori_loop(0, NT // 2, pair, cur)

        clab, ax, ay, az, ac = cur
        bi = lanebase + clab
        plsc.addupdate_scatter(table, [bi], ax)
        plsc.addupdate_scatter(table, [bi + KP], ay)
        plsc.addupdate_scatter(table, [bi + 2 * KP], az)
        plsc.addupdate_scatter(table, [bi + 3 * KP], ac)

        for c in range(4):
            def red_body(j, _, c=c):
                acc = zeros
                for l in range(LANES):
                    acc = acc + table[pl.ds(l * (4 * KP) + c * KP + j * 16, 16)]
                res[c, pl.ds(j * 16, 16)] = acc
                return 0

            lax.fori_loop(0, KP // 16, red_body, 0)

        pltpu.sync_copy(res, out_hbm.at[pl.ds(wid * 4, 4)])

    return k(dx, dy, dz, labels)


def _loss_body(*refs):
    o_ref = refs[-1]
    x = refs[0][...]
    for r in refs[1:-1]:
        x = x + r[...]
    rid = lax.broadcasted_iota(jnp.int32, (NW * 4, KP), 0)
    rmod = lax.rem(rid, 4)

    def csum(c):
        return jnp.sum(jnp.where(rmod == c, x, 0.0), axis=0, keepdims=True)

    s0, s1, s2, cnt = csum(0), csum(1), csum(2), csum(3)
    safe = jnp.where(cnt > 0.0, cnt, 1.0)
    d0, d1, d2 = s0 / safe, s1 / safe, s2 / safe
    pt = d0 * d0 + d1 * d1 + d2 * d2
    kidx = lax.broadcasted_iota(jnp.int32, (1, KP), 1)
    valid = kidx > 0
    contrib = (cnt >= 2.0) & valid
    present = (cnt >= 1.0) & valid
    total = jnp.sum(jnp.where(contrib, pt, 0.0))
    ntree = jnp.sum(jnp.where(present, 1.0, 0.0))
    loss = jnp.where(ntree > 0.0, total / jnp.maximum(ntree, 1.0), 0.0)
    o_ref[...] = jnp.full((1, 1), loss, jnp.float32)


def kernel(coords, offset_inst, offset_tree, tree_labels):
    del coords
    partials = []
    for h in range(H):
        lo = h * NPTS
        oi = jax.lax.slice_in_dim(offset_inst, lo, lo + NPTS, axis=0)
        ot = jax.lax.slice_in_dim(offset_tree, lo, lo + NPTS, axis=0)
        dx = oi[:, 0] - ot[:, 0]
        dy = oi[:, 1] - ot[:, 1]
        dz = oi[:, 2] - ot[:, 2]
        partials.append(_sc_segment_sums(dx, dy, dz, tree_labels, lo))
    loss = pl.pallas_call(
        _loss_body,
        out_shape=jax.ShapeDtypeStruct((1, 1), jnp.float32),
    )(*partials)
    return jnp.reshape(loss, ())


# speedup vs baseline: 1.0311x; 1.0183x over previous
import functools

import jax
import jax.numpy as jnp
from jax import lax
from jax.experimental import pallas as pl
from jax.experimental.pallas import tpu as pltpu
from jax.experimental.pallas import tpu_sc as plsc

N = 6400000
K = 1000
KP = 1024
NC = 2
NS = 16
NW = NC * NS
H = 1
NPTS = N // H
P = NPTS // NW
T = 4000
NT = P // T
SUB = T // 16
UF = 25
LANES = 16
TBL = LANES * 4 * KP


def _sc_segment_sums(dx, dy, dz, labels, base):
    mesh = plsc.VectorSubcoreMesh(core_axis_name="c", subcore_axis_name="s")

    @functools.partial(
        pl.kernel,
        mesh=mesh,
        out_type=jax.ShapeDtypeStruct((NW * 4, KP), jnp.float32),
        scratch_types=[
            pltpu.VMEM((TBL,), jnp.float32),
            pltpu.VMEM((4, KP), jnp.float32),
            pltpu.VMEM((T,), jnp.int32),
            pltpu.VMEM((T,), jnp.float32),
            pltpu.VMEM((T,), jnp.float32),
            pltpu.VMEM((T,), jnp.float32),
            pltpu.VMEM((T,), jnp.int32),
            pltpu.VMEM((T,), jnp.float32),
            pltpu.VMEM((T,), jnp.float32),
            pltpu.VMEM((T,), jnp.float32),
            pltpu.SemaphoreType.DMA,
            pltpu.SemaphoreType.DMA,
        ],
        compiler_params=pltpu.CompilerParams(needs_layout_passes=False),
    )
    def k(dx_hbm, dy_hbm, dz_hbm, lab_hbm, out_hbm,
          table, res, lab_v0, dx_v0, dy_v0, dz_v0,
          lab_v1, dx_v1, dy_v1, dz_v1, sem0, sem1):
        wid = lax.axis_index("s") * NC + lax.axis_index("c")
        lane = lax.iota(jnp.int32, 16)
        ivec = lane * SUB
        lanebase = lane * (4 * KP)
        zeros = jnp.zeros((16,), jnp.float32)
        ones = jnp.ones((16,), jnp.float32)
        izeros = jnp.zeros((16,), jnp.int32)
        bufs = ((lab_v0, dx_v0, dy_v0, dz_v0), (lab_v1, dx_v1, dy_v1, dz_v1))
        sems = (sem0, sem1)

        pbase = wid * P

        def start(t, b):
            off = pl.ds(pbase + t * T, T)
            loff = pl.ds(base + pbase + t * T, T)
            lab_b, dx_b, dy_b, dz_b = bufs[b]
            sem = sems[b]
            pltpu.async_copy(lab_hbm.at[loff], lab_b, sem)
            pltpu.async_copy(dx_hbm.at[off], dx_b, sem)
            pltpu.async_copy(dy_hbm.at[off], dy_b, sem)
            pltpu.async_copy(dz_hbm.at[off], dz_b, sem)

        cur = (izeros, zeros, zeros, zeros, zeros)
        start(0, 0)

        def zero_body(i, _):
            for u in range(8):
                table[pl.ds((i * 8 + u) * 16, 16)] = zeros
            return 0

        lax.fori_loop(0, TBL // 128, zero_body, 0)
        start(1, 1)

        def make_step(lab_b, dx_b, dy_b, dz_b):
            def step(s, carry):
                clab, ax, ay, az, ac = carry
                pidx = ivec + s
                nlab = plsc.load_gather(lab_b, [pidx])
                vx = plsc.load_gather(dx_b, [pidx])
                vy = plsc.load_gather(dy_b, [pidx])
                vz = plsc.load_gather(dz_b, [pidx])
                flush = nlab != clab
                bi = lanebase + clab
                plsc.addupdate_scatter(table, [bi], ax, mask=flush)
                plsc.addupdate_scatter(table, [bi + KP], ay, mask=flush)
                plsc.addupdate_scatter(table, [bi + 2 * KP], az, mask=flush)
                plsc.addupdate_scatter(table, [bi + 3 * KP], ac, mask=flush)
                ax = jnp.where(flush, vx, ax + vx)
                ay = jnp.where(flush, vy, ay + vy)
                az = jnp.where(flush, vz, az + vz)
                ac = jnp.where(flush, 1.0, ac + 1.0)
                return (nlab, ax, ay, az, ac)

            def stepu(i, carry):
                for u in range(UF):
                    carry = step(i * UF + u, carry)
                return carry

            return stepu

        def pair(i, carry):
            for b in range(2):
                t = 2 * i + b
                lab_b, dx_b, dy_b, dz_b = bufs[b]
                sem = sems[b]
                pltpu.make_async_copy(lab_hbm.at[pl.ds(0, T)], lab_b, sem).wait()
                pltpu.make_async_copy(dx_hbm.at[pl.ds(0, T)], dx_b, sem).wait()
                pltpu.make_async_copy(dy_hbm.at[pl.ds(0, T)], dy_b, sem).wait()
                pltpu.make_async_copy(dz_hbm.at[pl.ds(0, T)], dz_b, sem).wait()
                carry = lax.fori_loop(
                    0, SUB // UF, make_step(lab_b, dx_b, dy_b, dz_b), carry
                )

                @pl.when(t + 2 < NT)
                def _():
                    start(t + 2, b)

            return carry

        cur = lax.fori_loop(0, NT // 2, pair, cur)

        clab, ax, ay, az, ac = cur
        bi = lanebase + clab
        plsc.addupdate_scatter(table, [bi], ax)
        plsc.addupdate_scatter(table, [bi + KP], ay)
        plsc.addupdate_scatter(table, [bi + 2 * KP], az)
        plsc.addupdate_scatter(table, [bi + 3 * KP], ac)

        for c in range(4):
            def red_body(j, _, c=c):
                acc = zeros
                for l in range(LANES):
                    acc = acc + table[pl.ds(l * (4 * KP) + c * KP + j * 16, 16)]
                res[c, pl.ds(j * 16, 16)] = acc
                return 0

            lax.fori_loop(0, KP // 16, red_body, 0)

        pltpu.sync_copy(res, out_hbm.at[pl.ds(wid * 4, 4)])

    return k(dx, dy, dz, labels)


def _loss_body(*refs):
    o_ref = refs[-1]
    x = refs[0][...]
    for r in refs[1:-1]:
        x = x + r[...]
    rid = lax.broadcasted_iota(jnp.int32, (NW * 4, KP), 0)
    rmod = lax.rem(rid, 4)

    def csum(c):
        return jnp.sum(jnp.where(rmod == c, x, 0.0), axis=0, keepdims=True)

    s0, s1, s2, cnt = csum(0), csum(1), csum(2), csum(3)
    safe = jnp.where(cnt > 0.0, cnt, 1.0)
    d0, d1, d2 = s0 / safe, s1 / safe, s2 / safe
    pt = d0 * d0 + d1 * d1 + d2 * d2
    kidx = lax.broadcasted_iota(jnp.int32, (1, KP), 1)
    valid = kidx > 0
    contrib = (cnt >= 2.0) & valid
    present = (cnt >= 1.0) & valid
    total = jnp.sum(jnp.where(contrib, pt, 0.0))
    ntree = jnp.sum(jnp.where(present, 1.0, 0.0))
    loss = jnp.where(ntree > 0.0, total / jnp.maximum(ntree, 1.0), 0.0)
    o_ref[...] = jnp.full((1, 1), loss, jnp.float32)


def kernel(coords, offset_inst, offset_tree, tree_labels):
    del coords
    partials = []
    for h in range(H):
        lo = h * NPTS
        oi = jax.lax.slice_in_dim(offset_inst, lo, lo + NPTS, axis=0)
        ot = jax.lax.slice_in_dim(offset_tree, lo, lo + NPTS, axis=0)
        dx = oi[:, 0] - ot[:, 0]
        dy = oi[:, 1] - ot[:, 1]
        dz = oi[:, 2] - ot[:, 2]
        partials.append(_sc_segment_sums(dx, dy, dz, tree_labels, lo))
    loss = pl.pallas_call(
        _loss_body,
        out_shape=jax.ShapeDtypeStruct((1, 1), jnp.float32),
    )(*partials)
    return jnp.reshape(loss, ())
